# Initial kernel scaffold; baseline (speedup 1.0000x reference)
import jax
import jax.numpy as jnp
from jax.experimental import pallas as pl


def kernel(input, wg, w1, b1, w2, b2):
    raise NotImplementedError("write your pallas kernel here")


# speedup vs baseline: 1.7344x; 1.7344x over previous
import jax
import jax.numpy as jnp
from jax import lax
from jax.experimental import pallas as pl
from jax.experimental.pallas import tpu as pltpu
from jax.experimental.pallas import tpu_sc as plsc

E = 8
K = 2
D = 1024
N = 4096
C = 1024
A = K * N
NC, NS = 2, 16
NW = NC * NS
CHUNK = A // NW
WPE = C // CHUNK
GR = 64
TPB = N // NW
RT = 32
NEG = -1e30


def _bc(s):
    return lax.broadcast(s, (16,))


def _gate_body(x_ref, wg_ref, ids_ref, vals_ref):
    lg = jnp.dot(x_ref[...], wg_ref[...], preferred_element_type=jnp.float32)
    col = lax.broadcasted_iota(jnp.int32, lg.shape, 1)
    lg = jnp.where(col < E, lg, NEG)
    m1 = jnp.max(lg, axis=1, keepdims=True)
    i1 = jnp.min(jnp.where(lg >= m1, col, 128), axis=1, keepdims=True)
    lg2 = jnp.where(col == i1, NEG, lg)
    m2 = jnp.max(lg2, axis=1, keepdims=True)
    i2 = jnp.min(jnp.where(lg2 >= m2, col, 128), axis=1, keepdims=True)
    z = jnp.sum(jnp.where(col < E, jnp.exp(lg - m1), 0.0), axis=1, keepdims=True)
    v1 = 1.0 / z
    v2 = jnp.exp(m2 - m1) / z
    oc = lax.broadcasted_iota(jnp.int32, (N, 8), 1)
    ids_ref[...] = jnp.where(oc == 0, i1, jnp.where(oc == 1, i2, 0))
    vals_ref[...] = jnp.where(oc == 0, v1, jnp.where(oc == 1, v2, 0.0))


_gate = pl.pallas_call(
    _gate_body,
    out_shape=[
        jax.ShapeDtypeStruct((N, 8), jnp.int32),
        jax.ShapeDtypeStruct((N, 8), jnp.float32),
    ],
)


def _route_body(ids_hbm, vals_hbm, x_hbm, disp_hbm, dest_hbm, gates_hbm,
                ids_v, vals_v, dest_v, gates_v, src_v, rows_v, sem):
    wid = lax.axis_index("s") * NC + lax.axis_index("c")
    base = wid * CHUNK
    pltpu.sync_copy(ids_hbm, ids_v)
    pltpu.sync_copy(vals_hbm.at[pl.ds(base, CHUNK)], vals_v)
    iota = lax.iota(jnp.int32, 16)

    def _pref(v, hv):
        idsv = ids_v[pl.ds(v * 16, 16)]
        for e in range(E):
            cnt = jnp.sum(jnp.where(idsv == e, 1, 0))
            hv = jnp.where(iota == e, hv + _bc(cnt), hv)
        return hv

    hv = lax.fori_loop(0, wid * 16, _pref, jnp.zeros(16, jnp.int32))
    h = [jnp.sum(jnp.where(iota == e, hv, 0)) for e in range(E)]
    for v in range(CHUNK // 16):
        idsv = ids_v[pl.ds(base + v * 16, 16)]
        loc = jnp.zeros(16, jnp.int32)
        for e in range(E):
            m = idsv == e
            mi = jnp.where(m, 1, 0)
            cs = plsc.cumsum(mi)
            loc = jnp.where(m, cs - 1 + _bc(h[e]), loc)
            h[e] = h[e] + jnp.sum(mi)
        within = loc < C
        gates_v[pl.ds(v * 16, 16)] = jnp.where(within, vals_v[pl.ds(v * 16, 16)], 0.0)
        dest_v[pl.ds(v * 16, 16)] = idsv * C + jnp.minimum(loc, C - 1)
    pltpu.sync_copy(dest_v, dest_hbm.at[pl.ds(base, CHUNK)])
    pltpu.sync_copy(gates_v, gates_hbm.at[pl.ds(base, CHUNK)])

    em = wid // WPE
    lo = (wid % WPE) * CHUNK
    for v in range(CHUNK // 16):
        src_v[pl.ds(v * 16, 16)] = jnp.zeros(16, jnp.int32)

    def _slots(v, cnt):
        idsv = ids_v[pl.ds(v * 16, 16)]
        m = idsv == _bc(em)
        mi = jnp.where(m, 1, 0)
        pos = _bc(cnt) + plsc.cumsum(mi) - 1
        sel = m & (pos >= _bc(lo)) & (pos < _bc(lo + CHUNK))
        tok = (_bc(v * 16) + iota) & (N - 1)
        idx = jnp.clip(pos - lo, 0, CHUNK - 1)
        plsc.store_scatter(src_v, [idx], tok, mask=sel)
        return cnt + jnp.sum(mi)

    lax.fori_loop(0, A // 16, _slots, jnp.int32(0))

    slot0 = em * C + lo
    for r in range(CHUNK // GR):
        pltpu.async_copy(x_hbm.at[src_v.at[pl.ds(r * GR, GR)]], rows_v, sem).wait()
        pltpu.sync_copy(rows_v, disp_hbm.at[pl.ds(slot0 + r * GR, GR)])


_route = pl.kernel(
    _route_body,
    out_type=[
        jax.ShapeDtypeStruct((A, D), jnp.float32),
        jax.ShapeDtypeStruct((A,), jnp.int32),
        jax.ShapeDtypeStruct((A,), jnp.float32),
    ],
    mesh=plsc.VectorSubcoreMesh(core_axis_name="c", subcore_axis_name="s",
                                num_cores=NC, num_subcores=NS),
    compiler_params=pltpu.CompilerParams(needs_layout_passes=False),
    scratch_types=[
        pltpu.VMEM((A,), jnp.int32),
        pltpu.VMEM((CHUNK,), jnp.float32),
        pltpu.VMEM((CHUNK,), jnp.int32),
        pltpu.VMEM((CHUNK,), jnp.float32),
        pltpu.VMEM((CHUNK,), jnp.int32),
        pltpu.VMEM((GR, D), jnp.float32),
        pltpu.SemaphoreType.DMA,
    ],
)


CB = 512

def _ffn_body(disp_ref, w1_ref, b1_ref, w2_ref, b2_ref, y_ref):
    h = jnp.dot(disp_ref[0], w1_ref[0], preferred_element_type=jnp.float32)
    h = jnp.maximum(h + b1_ref[0], 0.0)
    y = jnp.dot(h, w2_ref[0], preferred_element_type=jnp.float32)
    y_ref[0] = y + b2_ref[0]


_ffn = pl.pallas_call(
    _ffn_body,
    grid=(E, C // CB),
    in_specs=[
        pl.BlockSpec((1, CB, D), lambda e, c: (e, c, 0)),
        pl.BlockSpec((1, D, D), lambda e, c: (e, 0, 0)),
        pl.BlockSpec((1, 1, D), lambda e, c: (e, 0, 0)),
        pl.BlockSpec((1, D, D), lambda e, c: (e, 0, 0)),
        pl.BlockSpec((1, 1, D), lambda e, c: (e, 0, 0)),
    ],
    out_specs=pl.BlockSpec((1, CB, D), lambda e, c: (e, c, 0)),
    out_shape=jax.ShapeDtypeStruct((E, C, D), jnp.float32),
)


def _combine_body(y_hbm, dest_hbm, gates_hbm, out_hbm,
                  d0_v, d1_v, g0_v, g1_v, rows0, rows1, ob, sem):
    wid = lax.axis_index("s") * NC + lax.axis_index("c")
    iota = lax.iota(jnp.int32, 16)
    for r in range(TPB // RT):
        base = wid * TPB + r * RT
        pltpu.sync_copy(dest_hbm.at[pl.ds(base, RT)], d0_v)
        pltpu.sync_copy(dest_hbm.at[pl.ds(N + base, RT)], d1_v)
        pltpu.sync_copy(gates_hbm.at[pl.ds(base, RT)], g0_v)
        pltpu.sync_copy(gates_hbm.at[pl.ds(N + base, RT)], g1_v)
        c0 = pltpu.async_copy(y_hbm.at[d0_v], rows0, sem)
        c1 = pltpu.async_copy(y_hbm.at[d1_v], rows1, sem)
        c0.wait()
        c1.wait()

        def _tok(t, _):
            ln = _bc(t & 15)
            g0 = jnp.sum(jnp.where(iota == ln, g0_v[pl.ds((t // 16) * 16, 16)], 0.0))
            g1 = jnp.sum(jnp.where(iota == ln, g1_v[pl.ds((t // 16) * 16, 16)], 0.0))
            g0v, g1v = _bc(g0), _bc(g1)

            def _j(j, __):
                ob[t, pl.ds(j * 16, 16)] = (g0v * rows0[t, pl.ds(j * 16, 16)]
                                            + g1v * rows1[t, pl.ds(j * 16, 16)])
                return 0

            lax.fori_loop(0, D // 16, _j, 0)
            return 0

        lax.fori_loop(0, RT, _tok, 0)
        pltpu.sync_copy(ob, out_hbm.at[pl.ds(base, RT)])


_combine = pl.kernel(
    _combine_body,
    out_type=jax.ShapeDtypeStruct((N, D), jnp.float32),
    mesh=plsc.VectorSubcoreMesh(core_axis_name="c", subcore_axis_name="s",
                                num_cores=NC, num_subcores=NS),
    compiler_params=pltpu.CompilerParams(needs_layout_passes=False),
    scratch_types=[
        pltpu.VMEM((RT,), jnp.int32),
        pltpu.VMEM((RT,), jnp.int32),
        pltpu.VMEM((RT,), jnp.float32),
        pltpu.VMEM((RT,), jnp.float32),
        pltpu.VMEM((RT, D), jnp.float32),
        pltpu.VMEM((RT, D), jnp.float32),
        pltpu.VMEM((RT, D), jnp.float32),
        pltpu.SemaphoreType.DMA,
    ],
)


def kernel(input, wg, w1, b1, w2, b2):
    x = input.astype(jnp.float32)
    wgp = jnp.zeros((D, 128), jnp.float32).at[:, :E].set(wg.T)
    ids8, vals8 = _gate(x, wgp)
    ids_sm = jnp.concatenate([ids8[:, 0], ids8[:, 1]])
    vals_sm = jnp.concatenate([vals8[:, 0], vals8[:, 1]])
    disp, dest, gates = _route(ids_sm, vals_sm, x)
    y = _ffn(disp.reshape(E, C, D), w1, b1, w2, b2)
    out = _combine(y.reshape(A, D), dest, gates)
    return out


# speedup vs baseline: 1.7379x; 1.0020x over previous
import jax
import jax.numpy as jnp
from jax import lax
from jax.experimental import pallas as pl
from jax.experimental.pallas import tpu as pltpu
from jax.experimental.pallas import tpu_sc as plsc

E = 8
K = 2
D = 1024
N = 4096
C = 1024
A = K * N
NC, NS = 2, 16
NW = NC * NS
CHUNK = A // NW
WPE = C // CHUNK
GR = 64
TPB = N // NW
RT = 32
NEG = -1e30


def _bc(s):
    return lax.broadcast(s, (16,))


def _gate_body(x_ref, wg_ref, ids_ref, vals_ref):
    lg = jnp.dot(x_ref[...], wg_ref[...], preferred_element_type=jnp.float32)
    col = lax.broadcasted_iota(jnp.int32, lg.shape, 1)
    lg = jnp.where(col < E, lg, NEG)
    m1 = jnp.max(lg, axis=1, keepdims=True)
    i1 = jnp.min(jnp.where(lg >= m1, col, 128), axis=1, keepdims=True)
    lg2 = jnp.where(col == i1, NEG, lg)
    m2 = jnp.max(lg2, axis=1, keepdims=True)
    i2 = jnp.min(jnp.where(lg2 >= m2, col, 128), axis=1, keepdims=True)
    z = jnp.sum(jnp.where(col < E, jnp.exp(lg - m1), 0.0), axis=1, keepdims=True)
    v1 = 1.0 / z
    v2 = jnp.exp(m2 - m1) / z
    oc = lax.broadcasted_iota(jnp.int32, (N, 8), 1)
    ids_ref[...] = jnp.where(oc == 0, i1, jnp.where(oc == 1, i2, 0))
    vals_ref[...] = jnp.where(oc == 0, v1, jnp.where(oc == 1, v2, 0.0))


_gate = pl.pallas_call(
    _gate_body,
    out_shape=[
        jax.ShapeDtypeStruct((N, 8), jnp.int32),
        jax.ShapeDtypeStruct((N, 8), jnp.float32),
    ],
)


def _route_body(ids_hbm, vals_hbm, x_hbm, disp_hbm, dest_hbm, gates_hbm,
                ids_v, vals_v, dest_v, gates_v, src_v, rows_v, sem):
    wid = lax.axis_index("s") * NC + lax.axis_index("c")
    base = wid * CHUNK
    pltpu.sync_copy(ids_hbm, ids_v)
    pltpu.sync_copy(vals_hbm.at[pl.ds(base, CHUNK)], vals_v)
    iota = lax.iota(jnp.int32, 16)

    def _pref(v, hv):
        idsv = ids_v[pl.ds(v * 16, 16)]
        for e in range(E):
            cnt = jnp.sum(jnp.where(idsv == e, 1, 0))
            hv = jnp.where(iota == e, hv + _bc(cnt), hv)
        return hv

    hv = lax.fori_loop(0, wid * 16, _pref, jnp.zeros(16, jnp.int32))
    h = [jnp.sum(jnp.where(iota == e, hv, 0)) for e in range(E)]
    for v in range(CHUNK // 16):
        idsv = ids_v[pl.ds(base + v * 16, 16)]
        loc = jnp.zeros(16, jnp.int32)
        for e in range(E):
            m = idsv == e
            mi = jnp.where(m, 1, 0)
            cs = plsc.cumsum(mi)
            loc = jnp.where(m, cs - 1 + _bc(h[e]), loc)
            h[e] = h[e] + jnp.sum(mi)
        within = loc < C
        gates_v[pl.ds(v * 16, 16)] = jnp.where(within, vals_v[pl.ds(v * 16, 16)], 0.0)
        dest_v[pl.ds(v * 16, 16)] = idsv * C + jnp.minimum(loc, C - 1)
    pltpu.sync_copy(dest_v, dest_hbm.at[pl.ds(base, CHUNK)])
    pltpu.sync_copy(gates_v, gates_hbm.at[pl.ds(base, CHUNK)])

    em = wid // WPE
    lo = (wid % WPE) * CHUNK
    for v in range(CHUNK // 16):
        src_v[pl.ds(v * 16, 16)] = jnp.zeros(16, jnp.int32)

    def _slots(v, cnt):
        idsv = ids_v[pl.ds(v * 16, 16)]
        m = idsv == _bc(em)
        mi = jnp.where(m, 1, 0)
        pos = _bc(cnt) + plsc.cumsum(mi) - 1
        sel = m & (pos >= _bc(lo)) & (pos < _bc(lo + CHUNK))
        tok = (_bc(v * 16) + iota) & (N - 1)
        idx = jnp.clip(pos - lo, 0, CHUNK - 1)
        plsc.store_scatter(src_v, [idx], tok, mask=sel)
        return cnt + jnp.sum(mi)

    lax.fori_loop(0, A // 16, _slots, jnp.int32(0))

    slot0 = em * C + lo
    for r in range(CHUNK // GR):
        pltpu.async_copy(x_hbm.at[src_v.at[pl.ds(r * GR, GR)]], rows_v, sem).wait()
        pltpu.sync_copy(rows_v, disp_hbm.at[pl.ds(slot0 + r * GR, GR)])


_route = pl.kernel(
    _route_body,
    out_type=[
        jax.ShapeDtypeStruct((A, D), jnp.float32),
        jax.ShapeDtypeStruct((A,), jnp.int32),
        jax.ShapeDtypeStruct((A,), jnp.float32),
    ],
    mesh=plsc.VectorSubcoreMesh(core_axis_name="c", subcore_axis_name="s",
                                num_cores=NC, num_subcores=NS),
    compiler_params=pltpu.CompilerParams(needs_layout_passes=False),
    scratch_types=[
        pltpu.VMEM((A,), jnp.int32),
        pltpu.VMEM((CHUNK,), jnp.float32),
        pltpu.VMEM((CHUNK,), jnp.int32),
        pltpu.VMEM((CHUNK,), jnp.float32),
        pltpu.VMEM((CHUNK,), jnp.int32),
        pltpu.VMEM((GR, D), jnp.float32),
        pltpu.SemaphoreType.DMA,
    ],
)


CB = 512

def _ffn_body(disp_ref, w1_ref, b1_ref, w2_ref, b2_ref, y_ref):
    a = disp_ref[0].astype(jnp.bfloat16)
    h = jnp.dot(a, w1_ref[0].astype(jnp.bfloat16), preferred_element_type=jnp.float32)
    h = jnp.maximum(h + b1_ref[0], 0.0)
    y = jnp.dot(h.astype(jnp.bfloat16), w2_ref[0].astype(jnp.bfloat16),
                preferred_element_type=jnp.float32)
    y_ref[0] = y + b2_ref[0]


_ffn = pl.pallas_call(
    _ffn_body,
    grid=(E, C // CB),
    in_specs=[
        pl.BlockSpec((1, CB, D), lambda e, c: (e, c, 0)),
        pl.BlockSpec((1, D, D), lambda e, c: (e, 0, 0)),
        pl.BlockSpec((1, 1, D), lambda e, c: (e, 0, 0)),
        pl.BlockSpec((1, D, D), lambda e, c: (e, 0, 0)),
        pl.BlockSpec((1, 1, D), lambda e, c: (e, 0, 0)),
    ],
    out_specs=pl.BlockSpec((1, CB, D), lambda e, c: (e, c, 0)),
    out_shape=jax.ShapeDtypeStruct((E, C, D), jnp.float32),
)


def _combine_body(y_hbm, dest_hbm, gates_hbm, out_hbm,
                  d0_v, d1_v, g0_v, g1_v, rows0, rows1, ob, sem):
    wid = lax.axis_index("s") * NC + lax.axis_index("c")
    iota = lax.iota(jnp.int32, 16)
    for r in range(TPB // RT):
        base = wid * TPB + r * RT
        pltpu.sync_copy(dest_hbm.at[pl.ds(base, RT)], d0_v)
        pltpu.sync_copy(dest_hbm.at[pl.ds(N + base, RT)], d1_v)
        pltpu.sync_copy(gates_hbm.at[pl.ds(base, RT)], g0_v)
        pltpu.sync_copy(gates_hbm.at[pl.ds(N + base, RT)], g1_v)
        c0 = pltpu.async_copy(y_hbm.at[d0_v], rows0, sem)
        c1 = pltpu.async_copy(y_hbm.at[d1_v], rows1, sem)
        c0.wait()
        c1.wait()

        def _tok(t, _):
            ln = _bc(t & 15)
            g0 = jnp.sum(jnp.where(iota == ln, g0_v[pl.ds((t // 16) * 16, 16)], 0.0))
            g1 = jnp.sum(jnp.where(iota == ln, g1_v[pl.ds((t // 16) * 16, 16)], 0.0))
            g0v, g1v = _bc(g0), _bc(g1)

            def _j(j, __):
                ob[t, pl.ds(j * 16, 16)] = (g0v * rows0[t, pl.ds(j * 16, 16)]
                                            + g1v * rows1[t, pl.ds(j * 16, 16)])
                return 0

            lax.fori_loop(0, D // 16, _j, 0)
            return 0

        lax.fori_loop(0, RT, _tok, 0)
        pltpu.sync_copy(ob, out_hbm.at[pl.ds(base, RT)])


_combine = pl.kernel(
    _combine_body,
    out_type=jax.ShapeDtypeStruct((N, D), jnp.float32),
    mesh=plsc.VectorSubcoreMesh(core_axis_name="c", subcore_axis_name="s",
                                num_cores=NC, num_subcores=NS),
    compiler_params=pltpu.CompilerParams(needs_layout_passes=False),
    scratch_types=[
        pltpu.VMEM((RT,), jnp.int32),
        pltpu.VMEM((RT,), jnp.int32),
        pltpu.VMEM((RT,), jnp.float32),
        pltpu.VMEM((RT,), jnp.float32),
        pltpu.VMEM((RT, D), jnp.float32),
        pltpu.VMEM((RT, D), jnp.float32),
        pltpu.VMEM((RT, D), jnp.float32),
        pltpu.SemaphoreType.DMA,
    ],
)


def kernel(input, wg, w1, b1, w2, b2):
    x = input.astype(jnp.float32)
    wgp = jnp.zeros((D, 128), jnp.float32).at[:, :E].set(wg.T)
    ids8, vals8 = _gate(x, wgp)
    ids_sm = jnp.concatenate([ids8[:, 0], ids8[:, 1]])
    vals_sm = jnp.concatenate([vals8[:, 0], vals8[:, 1]])
    disp, dest, gates = _route(ids_sm, vals_sm, x)
    y = _ffn(disp.reshape(E, C, D), w1, b1, w2, b2)
    out = _combine(y.reshape(A, D), dest, gates)
    return out
